# baseline (speedup 1.0000x reference)
import functools

import jax
import jax.numpy as jnp
from jax import lax
from jax.experimental import pallas as pl
from jax.experimental.pallas import tpu as pltpu
from jax.experimental.pallas import tpu_sc as plsc

_D = 256
_BN = 1000
_BE = 1000
_C = 40
_NSUB = 16
_NCORE = 2
_NW = _NSUB * _NCORE
_XW = 16
_TW = 256


def _pack2(lo, hi):
    lo_u = lax.bitcast_convert_type(lo.astype(jnp.bfloat16),
                                    jnp.uint16).astype(jnp.uint32)
    hi_u = lax.bitcast_convert_type(hi.astype(jnp.bfloat16),
                                    jnp.uint16).astype(jnp.uint32)
    return lax.bitcast_convert_type(lo_u | (hi_u << 16), jnp.float32)


def _unpack2(w):
    wu = lax.bitcast_convert_type(w, jnp.uint32)
    lo = lax.bitcast_convert_type((wu & 0xFFFF).astype(jnp.uint16),
                                  jnp.bfloat16).astype(jnp.float32)
    hi = lax.bitcast_convert_type((wu >> 16).astype(jnp.uint16),
                                  jnp.bfloat16).astype(jnp.float32)
    return lo, hi


def _silu(x):
    return x * jax.nn.sigmoid(x)


def _dot(a, b):
    return jnp.dot(a, b, preferred_element_type=jnp.float32)


def _prep_weights(params):
    gs = [('Wp_edge', 'bp_edge', 'blk_p'), ('Wl_edge', 'bl_edge', 'blk_l'),
          ('Wc_edge', 'bc_edge', 'blk_c')]
    ins = []
    for wk, bk, blk in gs:
        ins += [params[wk], params[bk].reshape(1, _D),
                params[blk]['We1'], params[blk]['be1'],
                params[blk]['bx'].reshape(2, 1),
                params[blk]['Wx'].reshape(2, 1, _D),
                params[blk]['be2']]

    def body(*refs):
        irefs, orefs = refs[:21], refs[21:]
        for g in range(3):
            (we_r, be_r, we1_r, be1_r, bx_r, wx_r,
             be2_r) = irefs[7 * g:7 * g + 7]
            o_r = orefs[g]
            for i in range(2):
                wmat = we1_r[i, 2 * _D + 1:, :]
                o_r[i, 0:16, :] = _dot(we_r[...], wmat)
                o_r[i, 16:17, :] = _dot(be_r[...], wmat) + be1_r[i:i + 1, :]
                o_r[i, 17:18, :] = we1_r[i, 2 * _D:2 * _D + 1, :]
                o_r[i, 18:19, :] = jnp.broadcast_to(bx_r[i:i + 1, :], (1, _D))
                o_r[i, 19:20, :] = wx_r[i]
                o_r[i, 20:21, :] = be2_r[i:i + 1, :]
                o_r[i, 21:24, :] = jnp.zeros((3, _D), jnp.float32)

    out_shape = [jax.ShapeDtypeStruct((2, 24, _D), jnp.float32)] * 3
    return pl.pallas_call(body, out_shape=out_shape)(*ins)


def _init_node(x, W2, ext2, split):
    N, F = x.shape

    def body(x_r, w_r, e_r, o_r):
        h = _dot(x_r[...], w_r[0]) + e_r[0, 0:1, :]
        mu = jnp.mean(h, axis=-1, keepdims=True)
        hm = h - mu
        v = jnp.mean(hm * hm, axis=-1, keepdims=True)
        o_r[...] = (hm * lax.rsqrt(v + 1e-5) * e_r[0, 1:2, :] +
                    e_r[0, 2:3, :])

    return pl.pallas_call(
        body,
        grid=(N // _BN,),
        in_specs=[pl.BlockSpec((_BN, F), lambda i: (i, 0)),
                  pl.BlockSpec((1, F, _D), lambda i: (i // split, 0, 0)),
                  pl.BlockSpec((1, 8, _D), lambda i: (i // split, 0, 0))],
        out_specs=pl.BlockSpec((_BN, _D), lambda i: (i, 0)),
        out_shape=jax.ShapeDtypeStruct((N, _D), jnp.float32),
    )(x, W2, ext2)


def _ab_prep(h, xpad, whd2, whs2, split):
    N = h.shape[0]

    def body(h_r, x_r, a_w, b_w, a_o, b_o):
        hv = h_r[...]
        xv = x_r[...]
        zx = jnp.zeros((_BN, 128 - _XW), jnp.float32)
        for o_r, w_r in ((a_o, a_w), (b_o, b_w)):
            av = _dot(hv, w_r[0])
            o_r[...] = jnp.concatenate(
                [_pack2(av[:, :128], av[:, 128:]), xv, zx], axis=-1)

    return pl.pallas_call(
        body,
        grid=(N // _BN,),
        in_specs=[pl.BlockSpec((_BN, _D), lambda i: (i, 0)),
                  pl.BlockSpec((_BN, _XW), lambda i: (i, 0)),
                  pl.BlockSpec((1, _D, _D), lambda i: (i // split, 0, 0)),
                  pl.BlockSpec((1, _D, _D), lambda i: (i // split, 0, 0))],
        out_specs=[pl.BlockSpec((_BN, _TW), lambda i: (i, 0))] * 2,
        out_shape=[jax.ShapeDtypeStruct((N, _TW), jnp.float32)] * 2,
    )(h, xpad, whd2, whs2)


def _edge_mlp(ga, gb, xd, xs, feats, wext2, we22, split, with_coef):
    E = ga.shape[0]

    def body(ga_r, gb_r, xd_r, xs_r, ft_r, wext_r, we2_r, *outs):
        ga0, ga1 = _unpack2(ga_r[...])
        gb0, gb1 = _unpack2(gb_r[...])
        rel = xd_r[...] - xs_r[...]
        d2 = jnp.sum(rel * rel, axis=-1, keepdims=True)
        wc = wext_r[0, 0:16, :]
        bc = wext_r[0, 16:17, :]
        wd2 = wext_r[0, 17:18, :]
        gsum = jnp.concatenate([ga0 + gb0, ga1 + gb1], axis=-1)
        pre = gsum + _dot(ft_r[...], wc) + bc + d2 * wd2
        m1 = _silu(pre)
        m = _silu(_dot(m1, we2_r[0]) + wext_r[0, 20:21, :])
        outs[0][0, :, :] = m[:, :128]
        outs[0][1, :, :] = m[:, 128:]
        if with_coef:
            bx = wext_r[0, 18:19, 0:1]
            wx = wext_r[0, 19:20, :]
            coef = jnp.sum(m * wx, axis=-1, keepdims=True) + bx
            outs[1][...] = jnp.concatenate(
                [rel * coef, jnp.zeros((_BE, 128 - _XW), jnp.float32)],
                axis=-1)

    out_shape = [jax.ShapeDtypeStruct((2, E, 128), jnp.float32)]
    out_specs = [pl.BlockSpec((2, _BE, 128), lambda i: (0, i, 0))]
    if with_coef:
        out_shape.append(jax.ShapeDtypeStruct((E, 128), jnp.float32))
        out_specs.append(pl.BlockSpec((_BE, 128), lambda i: (i, 0)))

    return pl.pallas_call(
        body,
        grid=(E // _BE,),
        in_specs=[pl.BlockSpec((_BE, 128), lambda i: (i, 0)),
                  pl.BlockSpec((_BE, 128), lambda i: (i, 0)),
                  pl.BlockSpec((_BE, _XW), lambda i: (i, 0)),
                  pl.BlockSpec((_BE, _XW), lambda i: (i, 0)),
                  pl.BlockSpec((_BE, 16), lambda i: (i, 0)),
                  pl.BlockSpec((1, 24, _D), lambda i: (i // split, 0, 0)),
                  pl.BlockSpec((1, _D, _D), lambda i: (i // split, 0, 0))],
        out_specs=out_specs,
        out_shape=out_shape,
    )(ga, gb, xd, xs, feats, wext2, we22)


def _node_update(h, ag2, w1a2, w1ba2, w1bb2, w22, next2, x, dx2, split,
                 with_x):
    N = h.shape[0]

    def body(*refs):
        if with_x:
            (h_r, ag_r, w1a_r, w1ba_r, w1bb_r, w2_r, ne_r,
             x_r, dx_r, ho_r, xo_r) = refs
        else:
            (h_r, ag_r, w1a_r, w1ba_r, w1bb_r, w2_r, ne_r, ho_r) = refs
        hv = h_r[...]
        t = (_dot(hv, w1a_r[0]) + _dot(ag_r[0, :, :], w1ba_r[0]) +
             _dot(ag_r[1, :, :], w1bb_r[0]) + ne_r[0, 0:1, :])
        t = _silu(t)
        ho_r[...] = hv + _dot(t, w2_r[0]) + ne_r[0, 1:2, :]
        if with_x:
            dx = (dx_r[0, :, :] + dx_r[1, :, :])[:, :_XW]
            xo_r[...] = x_r[...] + dx * (1.0 / 16.0)

    in_specs = [pl.BlockSpec((_BN, _D), lambda i: (i, 0)),
                pl.BlockSpec((2, _BN, 128), lambda i: (0, i, 0)),
                pl.BlockSpec((1, _D, _D), lambda i: (i // split, 0, 0)),
                pl.BlockSpec((1, 128, _D), lambda i: (i // split, 0, 0)),
                pl.BlockSpec((1, 128, _D), lambda i: (i // split, 0, 0)),
                pl.BlockSpec((1, _D, _D), lambda i: (i // split, 0, 0)),
                pl.BlockSpec((1, 8, _D), lambda i: (i // split, 0, 0))]
    out_specs = [pl.BlockSpec((_BN, _D), lambda i: (i, 0))]
    out_shape = [jax.ShapeDtypeStruct((N, _D), jnp.float32)]
    args = [h, ag2, w1a2, w1ba2, w1bb2, w22, next2]
    if with_x:
        in_specs += [pl.BlockSpec((_BN, _XW), lambda i: (i, 0)),
                     pl.BlockSpec((2, _BN, 128), lambda i: (0, i, 0))]
        out_specs.append(pl.BlockSpec((_BN, _XW), lambda i: (i, 0)))
        out_shape.append(jax.ShapeDtypeStruct((N, _XW), jnp.float32))
        args += [x, dx2]

    res = pl.pallas_call(
        body, grid=(N // _BN,), in_specs=in_specs, out_specs=out_specs,
        out_shape=out_shape)(*args)
    return res if with_x else (res[0], None)


def _sc_gather(td, ts, src, dst):
    E = src.shape[0]
    ept = E // _NW
    CG = 64
    nfull = ept // CG
    ctail = ept - nfull * CG
    npairs = nfull // 2
    rem = nfull % 2
    mesh = plsc.VectorSubcoreMesh(core_axis_name="c", subcore_axis_name="s")

    @functools.partial(
        pl.kernel, mesh=mesh,
        out_type=[jax.ShapeDtypeStruct((E, 128), jnp.float32),
                  jax.ShapeDtypeStruct((E, 128), jnp.float32),
                  jax.ShapeDtypeStruct((E, _XW), jnp.float32),
                  jax.ShapeDtypeStruct((E, _XW), jnp.float32)],
        scratch_types=[pltpu.VMEM((CG,), jnp.int32),
                       pltpu.VMEM((CG,), jnp.int32),
                       pltpu.VMEM((CG,), jnp.int32),
                       pltpu.VMEM((CG,), jnp.int32),
                       pltpu.VMEM((CG, _TW), jnp.float32),
                       pltpu.VMEM((CG, _TW), jnp.float32),
                       pltpu.VMEM((CG, _TW), jnp.float32),
                       pltpu.VMEM((CG, _TW), jnp.float32),
                       pltpu.VMEM((CG, _XW), jnp.float32),
                       pltpu.VMEM((CG, _XW), jnp.float32),
                       pltpu.VMEM((CG, _XW), jnp.float32),
                       pltpu.VMEM((CG, _XW), jnp.float32),
                       pltpu.SemaphoreType.DMA, pltpu.SemaphoreType.DMA,
                       pltpu.SemaphoreType.DMA, pltpu.SemaphoreType.DMA,
                       pltpu.SemaphoreType.DMA, pltpu.SemaphoreType.DMA])
    def k(a_hbm, b_hbm, src_hbm, dst_hbm, ga_hbm, gb_hbm, xd_hbm, xs_hbm,
          si0, di0, si1, di1, ab0, bb0, ab1, bb1,
          xd0, xs0, xd1, xs1,
          sa0, sb0, sa1, sb1, sw0, sw1):
        wid = lax.axis_index("s") * _NCORE + lax.axis_index("c")
        base = wid * ept

        def fetch(eb, n, si, di, ab, bb, sa, sb):
            sin = si.at[pl.ds(0, n)] if n != CG else si
            din = di.at[pl.ds(0, n)] if n != CG else di
            abn = ab.at[pl.ds(0, n)] if n != CG else ab
            bbn = bb.at[pl.ds(0, n)] if n != CG else bb
            pltpu.sync_copy(src_hbm.at[pl.ds(eb, n)], sin)
            pltpu.sync_copy(dst_hbm.at[pl.ds(eb, n)], din)
            ca = pltpu.async_copy(a_hbm.at[din], abn, sa)
            cb = pltpu.async_copy(b_hbm.at[sin], bbn, sb)
            return ca, cb

        def wback(eb, n, ab, bb, xdb, xsb, sw):
            @pl.loop(0, n)
            def _(r):
                xdb[r, pl.ds(0, _XW)] = ab[r, pl.ds(128, _XW)]
                xsb[r, pl.ds(0, _XW)] = bb[r, pl.ds(128, _XW)]

            es = pl.ds(eb, n)
            rs = pl.ds(0, n)
            xdn = xdb.at[rs] if n != CG else xdb
            xsn = xsb.at[rs] if n != CG else xsb
            return [
                pltpu.async_copy(ab.at[rs, pl.ds(0, 128)],
                                 ga_hbm.at[es], sw),
                pltpu.async_copy(bb.at[rs, pl.ds(0, 128)],
                                 gb_hbm.at[es], sw),
                pltpu.async_copy(xdn, xd_hbm.at[es], sw),
                pltpu.async_copy(xsn, xs_hbm.at[es], sw),
            ]

        @pl.loop(0, npairs)
        def _(j):
            e0 = base + j * (2 * CG)
            e1 = e0 + CG
            ca0, cb0 = fetch(e0, CG, si0, di0, ab0, bb0, sa0, sb0)
            ca1, cb1 = fetch(e1, CG, si1, di1, ab1, bb1, sa1, sb1)
            ca0.wait()
            cb0.wait()
            w0 = wback(e0, CG, ab0, bb0, xd0, xs0, sw0)
            ca1.wait()
            cb1.wait()
            w1 = wback(e1, CG, ab1, bb1, xd1, xs1, sw1)
            for w in w0 + w1:
                w.wait()

        tb = base + npairs * 2 * CG
        if rem:
            ca, cb = fetch(tb, CG, si0, di0, ab0, bb0, sa0, sb0)
            ca.wait()
            cb.wait()
            for w in wback(tb, CG, ab0, bb0, xd0, xs0, sw0):
                w.wait()
            tb = tb + CG
        if ctail:
            ca, cb = fetch(tb, ctail, si1, di1, ab1, bb1, sa1, sb1)
            ca.wait()
            cb.wait()
            for w in wback(tb, ctail, ab1, bb1, xd1, xs1, sw1):
                w.wait()

    return k(td, ts, src, dst)


def _sc_scatter_m(m2, dst, N):
    E = dst.shape[0]
    ept = E // _NSUB
    CS = 64
    nfull = ept // CS
    ctail = ept - nfull * CS
    npairs = nfull // 2
    rem = nfull % 2
    rpt = (N // _NSUB) // 8 * 8
    tail = N - _NSUB * rpt
    mesh = plsc.VectorSubcoreMesh(core_axis_name="c", subcore_axis_name="s")
    z128 = jnp.zeros((N, 128), jnp.float32)

    @functools.partial(
        pl.kernel, mesh=mesh,
        out_type=jax.ShapeDtypeStruct((2, N, 128), jnp.float32),
        scratch_types=[pltpu.VMEM((CS,), jnp.int32),
                       pltpu.VMEM((CS,), jnp.int32),
                       pltpu.VMEM((max(ctail, 8),), jnp.int32),
                       pltpu.VMEM((CS, 128), jnp.float32),
                       pltpu.VMEM((CS, 128), jnp.float32),
                       pltpu.VMEM((max(ctail, 8), 128), jnp.float32),
                       pltpu.VMEM_SHARED((N, 128), jnp.float32),
                       pltpu.SemaphoreType.DMA, pltpu.SemaphoreType.DMA,
                       pltpu.SemaphoreType.DMA, pltpu.SemaphoreType.DMA])
    def k(m_hbm, dst_hbm, z_hbm, agg_hbm,
          di0, di1, dit, rows0, rows1, rowst, acc, sr0, sr1, sd0, sd1):
        cid = lax.axis_index("c")
        sid = lax.axis_index("s")

        @pl.when(sid == 0)
        def _():
            pltpu.sync_copy(z_hbm, acc)

        plsc.subcore_barrier()
        base = sid * ept

        def fetch(eb, n, di, rows, sr):
            pltpu.sync_copy(dst_hbm.at[pl.ds(eb, n)], di)
            cr = pltpu.async_copy(m_hbm.at[cid, pl.ds(eb, n)], rows, sr)
            return cr, di, rows

        @pl.loop(0, npairs)
        def _(j):
            e0 = base + j * (2 * CS)
            c0, d0, r0 = fetch(e0, CS, di0, rows0, sr0)
            c1, d1, r1 = fetch(e0 + CS, CS, di1, rows1, sr1)
            c0.wait()
            a0 = pltpu.async_copy(r0, acc.at[d0], sd0, add=True)
            c1.wait()
            a1 = pltpu.async_copy(r1, acc.at[d1], sd1, add=True)
            a0.wait()
            a1.wait()

        tb = base + npairs * 2 * CS
        if rem:
            c0, d0, r0 = fetch(tb, CS, di0, rows0, sr0)
            c0.wait()
            pltpu.async_copy(r0, acc.at[d0], sd0, add=True).wait()
            tb = tb + CS
        if ctail:
            c1, d1, r1 = fetch(tb, ctail, dit, rowst, sr1)
            c1.wait()
            pltpu.async_copy(r1, acc.at[d1], sd1, add=True).wait()

        plsc.subcore_barrier()
        rb = sid * rpt
        pltpu.sync_copy(acc.at[pl.ds(rb, rpt)],
                        agg_hbm.at[cid, pl.ds(rb, rpt)])
        if tail:
            @pl.when(sid == 0)
            def _():
                tn = _NSUB * rpt
                pltpu.sync_copy(acc.at[pl.ds(tn, tail)],
                                agg_hbm.at[cid, pl.ds(tn, tail)])

    return k(m2, dst, z128)


def _sc_scatter_rc(rc, dst, N):
    E = dst.shape[0]
    eph = E // 2
    ept = eph // _NSUB
    CS = 64
    nfull = ept // CS
    ctail = ept - nfull * CS
    npairs = nfull // 2
    rem = nfull % 2
    rpt = (N // _NSUB) // 8 * 8
    tail = N - _NSUB * rpt
    mesh = plsc.VectorSubcoreMesh(core_axis_name="c", subcore_axis_name="s")
    z128 = jnp.zeros((N, 128), jnp.float32)

    @functools.partial(
        pl.kernel, mesh=mesh,
        out_type=jax.ShapeDtypeStruct((2, N, 128), jnp.float32),
        scratch_types=[pltpu.VMEM((CS,), jnp.int32),
                       pltpu.VMEM((CS,), jnp.int32),
                       pltpu.VMEM((max(ctail, 8),), jnp.int32),
                       pltpu.VMEM((CS, 128), jnp.float32),
                       pltpu.VMEM((CS, 128), jnp.float32),
                       pltpu.VMEM((max(ctail, 8), 128), jnp.float32),
                       pltpu.VMEM_SHARED((N, 128), jnp.float32),
                       pltpu.SemaphoreType.DMA, pltpu.SemaphoreType.DMA,
                       pltpu.SemaphoreType.DMA, pltpu.SemaphoreType.DMA])
    def k(rc_hbm, dst_hbm, z_hbm, dx_hbm,
          di0, di1, dit, rows0, rows1, rowst, acc, sr0, sr1, sd0, sd1):
        cid = lax.axis_index("c")
        sid = lax.axis_index("s")

        @pl.when(sid == 0)
        def _():
            pltpu.sync_copy(z_hbm, acc)

        plsc.subcore_barrier()
        base = cid * eph + sid * ept

        def fetch(eb, n, di, rows, sr):
            pltpu.sync_copy(dst_hbm.at[pl.ds(eb, n)], di)
            cr = pltpu.async_copy(rc_hbm.at[pl.ds(eb, n)], rows, sr)
            return cr, di, rows

        @pl.loop(0, npairs)
        def _(j):
            e0 = base + j * (2 * CS)
            c0, d0, r0 = fetch(e0, CS, di0, rows0, sr0)
            c1, d1, r1 = fetch(e0 + CS, CS, di1, rows1, sr1)
            c0.wait()
            a0 = pltpu.async_copy(r0, acc.at[d0], sd0, add=True)
            c1.wait()
            a1 = pltpu.async_copy(r1, acc.at[d1], sd1, add=True)
            a0.wait()
            a1.wait()

        tb = base + npairs * 2 * CS
        if rem:
            c0, d0, r0 = fetch(tb, CS, di0, rows0, sr0)
            c0.wait()
            pltpu.async_copy(r0, acc.at[d0], sd0, add=True).wait()
            tb = tb + CS
        if ctail:
            c1, d1, r1 = fetch(tb, ctail, dit, rowst, sr1)
            c1.wait()
            pltpu.async_copy(r1, acc.at[d1], sd1, add=True).wait()

        plsc.subcore_barrier()
        rb = sid * rpt
        pltpu.sync_copy(acc.at[pl.ds(rb, rpt)],
                        dx_hbm.at[cid, pl.ds(rb, rpt)])
        if tail:
            @pl.when(sid == 0)
            def _():
                tn = _NSUB * rpt
                pltpu.sync_copy(acc.at[pl.ds(tn, tail)],
                                dx_hbm.at[cid, pl.ds(tn, tail)])

    return k(rc, dst, z128)


def _egnn_block(h, xpad, feats, src, dst, stk, wext2, split_n, split_e,
                with_x):
    N = h.shape[0]
    td, ts = _ab_prep(h, xpad, stk['whd'], stk['whs'], split_n)
    ga, gb, xd, xs = _sc_gather(td, ts, src, dst)
    res = _edge_mlp(ga, gb, xd, xs, feats, wext2, stk['we2'], split_e,
                    with_coef=with_x)
    ag2 = _sc_scatter_m(res[0], dst, N)
    dx2 = _sc_scatter_rc(res[1], dst, N) if with_x else None
    return _node_update(h, ag2, stk['w1a'], stk['w1ba'], stk['w1bb'],
                        stk['w2'], stk['next'], xpad, dx2, split_n, with_x)


def _stack_layer(blks, i):
    z6 = jnp.zeros((6, _D), jnp.float32)
    return {
        'whd': jnp.stack([b['We1'][i, :_D, :] for b in blks]),
        'whs': jnp.stack([b['We1'][i, _D:2 * _D, :] for b in blks]),
        'we2': jnp.stack([b['We2'][i] for b in blks]),
        'w1a': jnp.stack([b['Wh1'][i, :_D, :] for b in blks]),
        'w1ba': jnp.stack([b['Wh1'][i, _D:_D + 128, :] for b in blks]),
        'w1bb': jnp.stack([b['Wh1'][i, _D + 128:, :] for b in blks]),
        'w2': jnp.stack([b['Wh2'][i] for b in blks]),
        'next': jnp.stack([jnp.concatenate(
            [b['bh1'][i][None], b['bh2'][i][None], z6]) for b in blks]),
    }


def kernel(xp, edge_index_p, ep_feats, coord_p, xl, edge_index_l, el_feats,
           coord_l, edge_index_c, ec_feats, coord_c, params):
    NP = xp.shape[0]
    L = params['blk_p']['We1'].shape[0]
    BIG = 1 << 20

    def pad_x(c):
        n = c.shape[0]
        return jnp.concatenate(
            [c, jnp.zeros((n, _XW - c.shape[1]), jnp.float32)], axis=1)

    sp, dp = (edge_index_p[0].astype(jnp.int32),
              edge_index_p[1].astype(jnp.int32))
    sl, dl = (edge_index_l[0].astype(jnp.int32),
              edge_index_l[1].astype(jnp.int32))
    sc, dc = (edge_index_c[0].astype(jnp.int32),
              edge_index_c[1].astype(jnp.int32))

    src_pl = jnp.concatenate([sp, sl + NP])
    dst_pl = jnp.concatenate([dp, dl + NP])
    feats_pl = jnp.concatenate([ep_feats, el_feats], axis=0)
    x2 = jnp.concatenate([xp, xl], axis=0)
    xpad_pl = jnp.concatenate([pad_x(coord_p), pad_x(coord_l)], axis=0)
    xpad_c = pad_x(coord_c)

    split_n = NP // _BN
    split_e = sp.shape[0] // _BE

    wext_p, wext_l, wext_c = _prep_weights(params)
    f32 = jnp.float32
    z5 = jnp.zeros((5, _D), f32)

    def init_ext(b, g, bb):
        return jnp.concatenate([b[None], g[None], bb[None], z5])

    w_init = jnp.stack([params['Wp_node'], params['Wl_node']])
    ext_init = jnp.stack([
        init_ext(params['bp_node'], params['ln_p_g'], params['ln_p_b']),
        init_ext(params['bl_node'], params['ln_l_g'], params['ln_l_b'])])
    hpl = _init_node(x2, w_init, ext_init, split_n)

    blks_pl = [params['blk_p'], params['blk_l']]
    blks_c = [params['blk_c']]

    for i in range(L):
        with_x = i < L - 1
        stk_pl = _stack_layer(blks_pl, i)
        stk_c = _stack_layer(blks_c, i)
        wext_pl_i = jnp.stack([wext_p[i], wext_l[i]])
        wext_c_i = wext_c[i][None]
        hpl, xpad_pl = _egnn_block(hpl, xpad_pl, feats_pl, src_pl, dst_pl,
                                   stk_pl, wext_pl_i, split_n, split_e,
                                   with_x)
        hpl, xpad_c = _egnn_block(hpl, xpad_c, ec_feats, sc, dc,
                                  stk_c, wext_c_i, BIG, BIG, with_x)

    return hpl[:NP], hpl[NP:], hpl


# speedup vs baseline: 1.0352x; 1.0352x over previous
import functools

import jax
import jax.numpy as jnp
from jax import lax
from jax.experimental import pallas as pl
from jax.experimental.pallas import tpu as pltpu
from jax.experimental.pallas import tpu_sc as plsc

_D = 256
_BN = 1000
_BE = 1000
_C = 40
_NSUB = 16
_NCORE = 2
_NW = _NSUB * _NCORE
_XW = 16
_TW = 256


def _pack2(lo, hi):
    lo_u = lax.bitcast_convert_type(lo.astype(jnp.bfloat16),
                                    jnp.uint16).astype(jnp.uint32)
    hi_u = lax.bitcast_convert_type(hi.astype(jnp.bfloat16),
                                    jnp.uint16).astype(jnp.uint32)
    return lax.bitcast_convert_type(lo_u | (hi_u << 16), jnp.float32)


def _unpack2(w):
    wu = lax.bitcast_convert_type(w, jnp.uint32)
    lo = lax.bitcast_convert_type((wu & 0xFFFF).astype(jnp.uint16),
                                  jnp.bfloat16).astype(jnp.float32)
    hi = lax.bitcast_convert_type((wu >> 16).astype(jnp.uint16),
                                  jnp.bfloat16).astype(jnp.float32)
    return lo, hi


def _silu(x):
    return x * jax.nn.sigmoid(x)


def _dot(a, b):
    return jnp.dot(a, b, preferred_element_type=jnp.float32)


def _prep_weights(params):
    gs = [('Wp_edge', 'bp_edge', 'blk_p'), ('Wl_edge', 'bl_edge', 'blk_l'),
          ('Wc_edge', 'bc_edge', 'blk_c')]
    ins = []
    for wk, bk, blk in gs:
        ins += [params[wk], params[bk].reshape(1, _D),
                params[blk]['We1'], params[blk]['be1'],
                params[blk]['bx'].reshape(2, 1),
                params[blk]['Wx'].reshape(2, 1, _D),
                params[blk]['be2']]

    def body(*refs):
        irefs, orefs = refs[:21], refs[21:]
        for g in range(3):
            (we_r, be_r, we1_r, be1_r, bx_r, wx_r,
             be2_r) = irefs[7 * g:7 * g + 7]
            o_r = orefs[g]
            for i in range(2):
                wmat = we1_r[i, 2 * _D + 1:, :]
                o_r[i, 0:16, :] = _dot(we_r[...], wmat)
                o_r[i, 16:17, :] = _dot(be_r[...], wmat) + be1_r[i:i + 1, :]
                o_r[i, 17:18, :] = we1_r[i, 2 * _D:2 * _D + 1, :]
                o_r[i, 18:19, :] = jnp.broadcast_to(bx_r[i:i + 1, :], (1, _D))
                o_r[i, 19:20, :] = wx_r[i]
                o_r[i, 20:21, :] = be2_r[i:i + 1, :]
                o_r[i, 21:24, :] = jnp.zeros((3, _D), jnp.float32)

    out_shape = [jax.ShapeDtypeStruct((2, 24, _D), jnp.float32)] * 3
    return pl.pallas_call(body, out_shape=out_shape)(*ins)


def _init_node(x, W2, ext2, split):
    N, F = x.shape

    def body(x_r, w_r, e_r, o_r):
        h = _dot(x_r[...], w_r[0]) + e_r[0, 0:1, :]
        mu = jnp.mean(h, axis=-1, keepdims=True)
        hm = h - mu
        v = jnp.mean(hm * hm, axis=-1, keepdims=True)
        o_r[...] = (hm * lax.rsqrt(v + 1e-5) * e_r[0, 1:2, :] +
                    e_r[0, 2:3, :])

    return pl.pallas_call(
        body,
        grid=(N // _BN,),
        in_specs=[pl.BlockSpec((_BN, F), lambda i: (i, 0)),
                  pl.BlockSpec((1, F, _D), lambda i: (i // split, 0, 0)),
                  pl.BlockSpec((1, 8, _D), lambda i: (i // split, 0, 0))],
        out_specs=pl.BlockSpec((_BN, _D), lambda i: (i, 0)),
        out_shape=jax.ShapeDtypeStruct((N, _D), jnp.float32),
    )(x, W2, ext2)


def _ab_prep(h, xpad, whd2, whs2, split):
    N = h.shape[0]

    def body(h_r, x_r, a_w, b_w, a_o, b_o):
        hv = h_r[...]
        xv = x_r[...]
        zx = jnp.zeros((_BN, 128 - _XW), jnp.float32)
        for o_r, w_r in ((a_o, a_w), (b_o, b_w)):
            av = _dot(hv, w_r[0])
            o_r[...] = jnp.concatenate(
                [_pack2(av[:, :128], av[:, 128:]), xv, zx], axis=-1)

    return pl.pallas_call(
        body,
        grid=(N // _BN,),
        in_specs=[pl.BlockSpec((_BN, _D), lambda i: (i, 0)),
                  pl.BlockSpec((_BN, _XW), lambda i: (i, 0)),
                  pl.BlockSpec((1, _D, _D), lambda i: (i // split, 0, 0)),
                  pl.BlockSpec((1, _D, _D), lambda i: (i // split, 0, 0))],
        out_specs=[pl.BlockSpec((_BN, _TW), lambda i: (i, 0))] * 2,
        out_shape=[jax.ShapeDtypeStruct((N, _TW), jnp.float32)] * 2,
    )(h, xpad, whd2, whs2)


def _edge_mlp(ga, gb, xd, xs, feats, wext2, we22, split, with_coef):
    E = ga.shape[0]

    def body(ga_r, gb_r, xd_r, xs_r, ft_r, wext_r, we2_r, *outs):
        ga0, ga1 = _unpack2(ga_r[...])
        gb0, gb1 = _unpack2(gb_r[...])
        rel = xd_r[...] - xs_r[...]
        d2 = jnp.sum(rel * rel, axis=-1, keepdims=True)
        wc = wext_r[0, 0:16, :]
        bc = wext_r[0, 16:17, :]
        wd2 = wext_r[0, 17:18, :]
        gsum = jnp.concatenate([ga0 + gb0, ga1 + gb1], axis=-1)
        pre = gsum + _dot(ft_r[...], wc) + bc + d2 * wd2
        m1 = _silu(pre)
        m = _silu(_dot(m1, we2_r[0]) + wext_r[0, 20:21, :])
        outs[0][0, :, :] = m[:, :128]
        outs[0][1, :, :] = m[:, 128:]
        if with_coef:
            bx = wext_r[0, 18:19, 0:1]
            wx = wext_r[0, 19:20, :]
            coef = jnp.sum(m * wx, axis=-1, keepdims=True) + bx
            outs[1][...] = jnp.concatenate(
                [rel * coef, jnp.zeros((_BE, 128 - _XW), jnp.float32)],
                axis=-1)

    out_shape = [jax.ShapeDtypeStruct((2, E, 128), jnp.float32)]
    out_specs = [pl.BlockSpec((2, _BE, 128), lambda i: (0, i, 0))]
    if with_coef:
        out_shape.append(jax.ShapeDtypeStruct((E, 128), jnp.float32))
        out_specs.append(pl.BlockSpec((_BE, 128), lambda i: (i, 0)))

    return pl.pallas_call(
        body,
        grid=(E // _BE,),
        in_specs=[pl.BlockSpec((_BE, 128), lambda i: (i, 0)),
                  pl.BlockSpec((_BE, 128), lambda i: (i, 0)),
                  pl.BlockSpec((_BE, _XW), lambda i: (i, 0)),
                  pl.BlockSpec((_BE, _XW), lambda i: (i, 0)),
                  pl.BlockSpec((_BE, 16), lambda i: (i, 0)),
                  pl.BlockSpec((1, 24, _D), lambda i: (i // split, 0, 0)),
                  pl.BlockSpec((1, _D, _D), lambda i: (i // split, 0, 0))],
        out_specs=out_specs,
        out_shape=out_shape,
    )(ga, gb, xd, xs, feats, wext2, we22)


def _node_update(h, ag2, w1a2, w1ba2, w1bb2, w22, next2, x, dx2, split,
                 with_x):
    N = h.shape[0]

    def body(*refs):
        if with_x:
            (h_r, ag_r, w1a_r, w1ba_r, w1bb_r, w2_r, ne_r,
             x_r, dx_r, ho_r, xo_r) = refs
        else:
            (h_r, ag_r, w1a_r, w1ba_r, w1bb_r, w2_r, ne_r, ho_r) = refs
        hv = h_r[...]
        t = (_dot(hv, w1a_r[0]) + _dot(ag_r[0, :, :], w1ba_r[0]) +
             _dot(ag_r[1, :, :], w1bb_r[0]) + ne_r[0, 0:1, :])
        t = _silu(t)
        ho_r[...] = hv + _dot(t, w2_r[0]) + ne_r[0, 1:2, :]
        if with_x:
            dx = (dx_r[0, :, :] + dx_r[1, :, :])[:, :_XW]
            xo_r[...] = x_r[...] + dx * (1.0 / 16.0)

    in_specs = [pl.BlockSpec((_BN, _D), lambda i: (i, 0)),
                pl.BlockSpec((2, _BN, 128), lambda i: (0, i, 0)),
                pl.BlockSpec((1, _D, _D), lambda i: (i // split, 0, 0)),
                pl.BlockSpec((1, 128, _D), lambda i: (i // split, 0, 0)),
                pl.BlockSpec((1, 128, _D), lambda i: (i // split, 0, 0)),
                pl.BlockSpec((1, _D, _D), lambda i: (i // split, 0, 0)),
                pl.BlockSpec((1, 8, _D), lambda i: (i // split, 0, 0))]
    out_specs = [pl.BlockSpec((_BN, _D), lambda i: (i, 0))]
    out_shape = [jax.ShapeDtypeStruct((N, _D), jnp.float32)]
    args = [h, ag2, w1a2, w1ba2, w1bb2, w22, next2]
    if with_x:
        in_specs += [pl.BlockSpec((_BN, _XW), lambda i: (i, 0)),
                     pl.BlockSpec((2, _BN, 128), lambda i: (0, i, 0))]
        out_specs.append(pl.BlockSpec((_BN, _XW), lambda i: (i, 0)))
        out_shape.append(jax.ShapeDtypeStruct((N, _XW), jnp.float32))
        args += [x, dx2]

    res = pl.pallas_call(
        body, grid=(N // _BN,), in_specs=in_specs, out_specs=out_specs,
        out_shape=out_shape)(*args)
    return res if with_x else (res[0], None)


def _sc_gather(td, ts, src, dst):
    E = src.shape[0]
    ept = E // _NW
    CG = 64
    nfull = ept // CG
    ctail = ept - nfull * CG
    npairs = nfull // 2
    rem = nfull % 2
    mesh = plsc.VectorSubcoreMesh(core_axis_name="c", subcore_axis_name="s")

    @functools.partial(
        pl.kernel, mesh=mesh,
        out_type=[jax.ShapeDtypeStruct((E, 128), jnp.float32),
                  jax.ShapeDtypeStruct((E, 128), jnp.float32),
                  jax.ShapeDtypeStruct((E, _XW), jnp.float32),
                  jax.ShapeDtypeStruct((E, _XW), jnp.float32)],
        scratch_types=[pltpu.VMEM((CG,), jnp.int32),
                       pltpu.VMEM((CG,), jnp.int32),
                       pltpu.VMEM((CG,), jnp.int32),
                       pltpu.VMEM((CG,), jnp.int32),
                       pltpu.VMEM((CG, _TW), jnp.float32),
                       pltpu.VMEM((CG, _TW), jnp.float32),
                       pltpu.VMEM((CG, _TW), jnp.float32),
                       pltpu.VMEM((CG, _TW), jnp.float32),
                       pltpu.VMEM((CG, _XW), jnp.float32),
                       pltpu.VMEM((CG, _XW), jnp.float32),
                       pltpu.VMEM((CG, _XW), jnp.float32),
                       pltpu.VMEM((CG, _XW), jnp.float32),
                       pltpu.SemaphoreType.DMA, pltpu.SemaphoreType.DMA,
                       pltpu.SemaphoreType.DMA, pltpu.SemaphoreType.DMA,
                       pltpu.SemaphoreType.DMA, pltpu.SemaphoreType.DMA])
    def k(a_hbm, b_hbm, src_hbm, dst_hbm, ga_hbm, gb_hbm, xd_hbm, xs_hbm,
          si0, di0, si1, di1, ab0, bb0, ab1, bb1,
          xd0, xs0, xd1, xs1,
          sa0, sb0, sa1, sb1, sw0, sw1):
        wid = lax.axis_index("s") * _NCORE + lax.axis_index("c")
        base = wid * ept

        def fetch(eb, n, si, di, ab, bb, sa, sb):
            sin = si.at[pl.ds(0, n)] if n != CG else si
            din = di.at[pl.ds(0, n)] if n != CG else di
            abn = ab.at[pl.ds(0, n)] if n != CG else ab
            bbn = bb.at[pl.ds(0, n)] if n != CG else bb
            pltpu.sync_copy(src_hbm.at[pl.ds(eb, n)], sin)
            pltpu.sync_copy(dst_hbm.at[pl.ds(eb, n)], din)
            ca = pltpu.async_copy(a_hbm.at[din], abn, sa)
            cb = pltpu.async_copy(b_hbm.at[sin], bbn, sb)
            return ca, cb

        def wback(eb, n, ab, bb, xdb, xsb, sw):
            @pl.loop(0, n)
            def _(r):
                xdb[r, pl.ds(0, _XW)] = ab[r, pl.ds(128, _XW)]
                xsb[r, pl.ds(0, _XW)] = bb[r, pl.ds(128, _XW)]

            es = pl.ds(eb, n)
            rs = pl.ds(0, n)
            xdn = xdb.at[rs] if n != CG else xdb
            xsn = xsb.at[rs] if n != CG else xsb
            return [
                pltpu.async_copy(ab.at[rs, pl.ds(0, 128)],
                                 ga_hbm.at[es], sw),
                pltpu.async_copy(bb.at[rs, pl.ds(0, 128)],
                                 gb_hbm.at[es], sw),
                pltpu.async_copy(xdn, xd_hbm.at[es], sw),
                pltpu.async_copy(xsn, xs_hbm.at[es], sw),
            ]

        @pl.loop(0, npairs)
        def _(j):
            e0 = base + j * (2 * CG)
            e1 = e0 + CG
            ca0, cb0 = fetch(e0, CG, si0, di0, ab0, bb0, sa0, sb0)
            ca1, cb1 = fetch(e1, CG, si1, di1, ab1, bb1, sa1, sb1)
            ca0.wait()
            cb0.wait()
            w0 = wback(e0, CG, ab0, bb0, xd0, xs0, sw0)
            ca1.wait()
            cb1.wait()
            w1 = wback(e1, CG, ab1, bb1, xd1, xs1, sw1)
            for w in w0 + w1:
                w.wait()

        tb = base + npairs * 2 * CG
        if rem:
            ca, cb = fetch(tb, CG, si0, di0, ab0, bb0, sa0, sb0)
            ca.wait()
            cb.wait()
            for w in wback(tb, CG, ab0, bb0, xd0, xs0, sw0):
                w.wait()
            tb = tb + CG
        if ctail:
            ca, cb = fetch(tb, ctail, si1, di1, ab1, bb1, sa1, sb1)
            ca.wait()
            cb.wait()
            for w in wback(tb, ctail, ab1, bb1, xd1, xs1, sw1):
                w.wait()

    return k(td, ts, src, dst)


def _sc_scatter_m(m2, dst, N):
    E = dst.shape[0]
    ept = E // _NSUB
    CS = 64
    nfull = ept // CS
    ctail = ept - nfull * CS
    npairs = nfull // 2
    rem = nfull % 2
    rpt = (N // _NSUB) // 8 * 8
    tail = N - _NSUB * rpt
    mesh = plsc.VectorSubcoreMesh(core_axis_name="c", subcore_axis_name="s")
    z128 = jnp.zeros((N, 128), jnp.float32)

    @functools.partial(
        pl.kernel, mesh=mesh,
        out_type=jax.ShapeDtypeStruct((2, N, 128), jnp.float32),
        scratch_types=[pltpu.VMEM((CS,), jnp.int32),
                       pltpu.VMEM((CS,), jnp.int32),
                       pltpu.VMEM((max(ctail, 8),), jnp.int32),
                       pltpu.VMEM((CS, 128), jnp.float32),
                       pltpu.VMEM((CS, 128), jnp.float32),
                       pltpu.VMEM((max(ctail, 8), 128), jnp.float32),
                       pltpu.VMEM_SHARED((N, 128), jnp.float32),
                       pltpu.SemaphoreType.DMA, pltpu.SemaphoreType.DMA,
                       pltpu.SemaphoreType.DMA, pltpu.SemaphoreType.DMA])
    def k(m_hbm, dst_hbm, z_hbm, agg_hbm,
          di0, di1, dit, rows0, rows1, rowst, acc, sr0, sr1, sd0, sd1):
        cid = lax.axis_index("c")
        sid = lax.axis_index("s")

        @pl.when(sid == 0)
        def _():
            pltpu.sync_copy(z_hbm, acc)

        plsc.subcore_barrier()
        base = sid * ept

        def fetch(eb, n, di, rows, sr):
            pltpu.sync_copy(dst_hbm.at[pl.ds(eb, n)], di)
            cr = pltpu.async_copy(m_hbm.at[cid, pl.ds(eb, n)], rows, sr)
            return cr, di, rows

        @pl.loop(0, npairs)
        def _(j):
            e0 = base + j * (2 * CS)
            c0, d0, r0 = fetch(e0, CS, di0, rows0, sr0)
            c1, d1, r1 = fetch(e0 + CS, CS, di1, rows1, sr1)
            c0.wait()
            a0 = pltpu.async_copy(r0, acc.at[d0], sd0, add=True)
            c1.wait()
            a1 = pltpu.async_copy(r1, acc.at[d1], sd1, add=True)
            a0.wait()
            a1.wait()

        tb = base + npairs * 2 * CS
        if rem:
            c0, d0, r0 = fetch(tb, CS, di0, rows0, sr0)
            c0.wait()
            pltpu.async_copy(r0, acc.at[d0], sd0, add=True).wait()
            tb = tb + CS
        if ctail:
            c1, d1, r1 = fetch(tb, ctail, dit, rowst, sr1)
            c1.wait()
            pltpu.async_copy(r1, acc.at[d1], sd1, add=True).wait()

        plsc.subcore_barrier()
        rb = sid * rpt
        pltpu.sync_copy(acc.at[pl.ds(rb, rpt)],
                        agg_hbm.at[cid, pl.ds(rb, rpt)])
        if tail:
            @pl.when(sid == 0)
            def _():
                tn = _NSUB * rpt
                pltpu.sync_copy(acc.at[pl.ds(tn, tail)],
                                agg_hbm.at[cid, pl.ds(tn, tail)])

    return k(m2, dst, z128)


def _sc_scatter_rc(rc, dst, N):
    E = dst.shape[0]
    eph = E // 2
    ept = eph // _NSUB
    CS = 64
    nfull = ept // CS
    ctail = ept - nfull * CS
    npairs = nfull // 2
    rem = nfull % 2
    rpt = (N // _NSUB) // 8 * 8
    tail = N - _NSUB * rpt
    mesh = plsc.VectorSubcoreMesh(core_axis_name="c", subcore_axis_name="s")
    z128 = jnp.zeros((N, 128), jnp.float32)

    @functools.partial(
        pl.kernel, mesh=mesh,
        out_type=jax.ShapeDtypeStruct((2, N, 128), jnp.float32),
        scratch_types=[pltpu.VMEM((CS,), jnp.int32),
                       pltpu.VMEM((CS,), jnp.int32),
                       pltpu.VMEM((max(ctail, 8),), jnp.int32),
                       pltpu.VMEM((CS, 128), jnp.float32),
                       pltpu.VMEM((CS, 128), jnp.float32),
                       pltpu.VMEM((max(ctail, 8), 128), jnp.float32),
                       pltpu.VMEM_SHARED((N, 128), jnp.float32),
                       pltpu.SemaphoreType.DMA, pltpu.SemaphoreType.DMA,
                       pltpu.SemaphoreType.DMA, pltpu.SemaphoreType.DMA])
    def k(rc_hbm, dst_hbm, z_hbm, dx_hbm,
          di0, di1, dit, rows0, rows1, rowst, acc, sr0, sr1, sd0, sd1):
        cid = lax.axis_index("c")
        sid = lax.axis_index("s")

        @pl.when(sid == 0)
        def _():
            pltpu.sync_copy(z_hbm, acc)

        plsc.subcore_barrier()
        base = cid * eph + sid * ept

        def fetch(eb, n, di, rows, sr):
            pltpu.sync_copy(dst_hbm.at[pl.ds(eb, n)], di)
            cr = pltpu.async_copy(rc_hbm.at[pl.ds(eb, n)], rows, sr)
            return cr, di, rows

        @pl.loop(0, npairs)
        def _(j):
            e0 = base + j * (2 * CS)
            c0, d0, r0 = fetch(e0, CS, di0, rows0, sr0)
            c1, d1, r1 = fetch(e0 + CS, CS, di1, rows1, sr1)
            c0.wait()
            a0 = pltpu.async_copy(r0, acc.at[d0], sd0, add=True)
            c1.wait()
            a1 = pltpu.async_copy(r1, acc.at[d1], sd1, add=True)
            a0.wait()
            a1.wait()

        tb = base + npairs * 2 * CS
        if rem:
            c0, d0, r0 = fetch(tb, CS, di0, rows0, sr0)
            c0.wait()
            pltpu.async_copy(r0, acc.at[d0], sd0, add=True).wait()
            tb = tb + CS
        if ctail:
            c1, d1, r1 = fetch(tb, ctail, dit, rowst, sr1)
            c1.wait()
            pltpu.async_copy(r1, acc.at[d1], sd1, add=True).wait()

        plsc.subcore_barrier()
        rb = sid * rpt
        pltpu.sync_copy(acc.at[pl.ds(rb, rpt)],
                        dx_hbm.at[cid, pl.ds(rb, rpt)])
        if tail:
            @pl.when(sid == 0)
            def _():
                tn = _NSUB * rpt
                pltpu.sync_copy(acc.at[pl.ds(tn, tail)],
                                dx_hbm.at[cid, pl.ds(tn, tail)])

    return k(rc, dst, z128)


def _egnn_block(h, xpad, feats, src, dst, stk, wext2, split_n, split_e,
                with_x):
    N = h.shape[0]
    td, ts = _ab_prep(h, xpad, stk['whd'], stk['whs'], split_n)
    ga, gb, xd, xs = _sc_gather(td, ts, src, dst)
    res = _edge_mlp(ga, gb, xd, xs, feats, wext2, stk['we2'], split_e,
                    with_coef=with_x)
    ag2 = _sc_scatter_m(res[0], dst, N)
    dx2 = _sc_scatter_rc(res[1], dst, N) if with_x else None
    return _node_update(h, ag2, stk['w1a'], stk['w1ba'], stk['w1bb'],
                        stk['w2'], stk['next'], xpad, dx2, split_n, with_x)


def _stack_layer(blks, i):
    z6 = jnp.zeros((6, _D), jnp.float32)
    return {
        'whd': jnp.stack([b['We1'][i, :_D, :] for b in blks]),
        'whs': jnp.stack([b['We1'][i, _D:2 * _D, :] for b in blks]),
        'we2': jnp.stack([b['We2'][i] for b in blks]),
        'w1a': jnp.stack([b['Wh1'][i, :_D, :] for b in blks]),
        'w1ba': jnp.stack([b['Wh1'][i, _D:_D + 128, :] for b in blks]),
        'w1bb': jnp.stack([b['Wh1'][i, _D + 128:, :] for b in blks]),
        'w2': jnp.stack([b['Wh2'][i] for b in blks]),
        'next': jnp.stack([jnp.concatenate(
            [b['bh1'][i][None], b['bh2'][i][None], z6]) for b in blks]),
    }


def kernel(xp, edge_index_p, ep_feats, coord_p, xl, edge_index_l, el_feats,
           coord_l, edge_index_c, ec_feats, coord_c, params):
    NP = xp.shape[0]
    L = params['blk_p']['We1'].shape[0]
    BIG = 1 << 20

    def pad_x(c):
        n = c.shape[0]
        return jnp.concatenate(
            [c, jnp.zeros((n, _XW - c.shape[1]), jnp.float32)], axis=1)

    sp, dp = (edge_index_p[0].astype(jnp.int32),
              edge_index_p[1].astype(jnp.int32))
    sl, dl = (edge_index_l[0].astype(jnp.int32),
              edge_index_l[1].astype(jnp.int32))
    sc, dc = (edge_index_c[0].astype(jnp.int32),
              edge_index_c[1].astype(jnp.int32))

    xpp, xpl, xpc = pad_x(coord_p), pad_x(coord_l), pad_x(coord_c)

    wext_p, wext_l, wext_c = _prep_weights(params)
    z5 = jnp.zeros((5, _D), jnp.float32)

    def init_ext(b, g, bb):
        return jnp.concatenate([b[None], g[None], bb[None], z5])

    hp = _init_node(xp, params['Wp_node'][None],
                    init_ext(params['bp_node'], params['ln_p_g'],
                             params['ln_p_b'])[None], BIG)
    hl = _init_node(xl, params['Wl_node'][None],
                    init_ext(params['bl_node'], params['ln_l_g'],
                             params['ln_l_b'])[None], BIG)

    for i in range(L):
        with_x = i < L - 1
        stk_p = _stack_layer([params['blk_p']], i)
        stk_l = _stack_layer([params['blk_l']], i)
        stk_c = _stack_layer([params['blk_c']], i)
        hp, xpp = _egnn_block(hp, xpp, ep_feats, sp, dp, stk_p,
                              wext_p[i][None], BIG, BIG, with_x)
        hl, xpl = _egnn_block(hl, xpl, el_feats, sl, dl, stk_l,
                              wext_l[i][None], BIG, BIG, with_x)
        hc = jnp.concatenate([hp, hl], axis=0)
        hc, xpc = _egnn_block(hc, xpc, ec_feats, sc, dc, stk_c,
                              wext_c[i][None], BIG, BIG, with_x)
        hp = hc[:NP]
        hl = hc[NP:]

    return hp, hl, hc


# speedup vs baseline: 1.1980x; 1.1573x over previous
import functools

import jax
import jax.numpy as jnp
from jax import lax
from jax.experimental import pallas as pl
from jax.experimental.pallas import tpu as pltpu
from jax.experimental.pallas import tpu_sc as plsc

_D = 256
_BN = 1000
_BE = 1000
_C = 40
_NSUB = 16
_NCORE = 2
_NW = _NSUB * _NCORE
_XW = 16
_TW = 256


def _pack2(lo, hi):
    lo_u = lax.bitcast_convert_type(lo.astype(jnp.bfloat16),
                                    jnp.uint16).astype(jnp.uint32)
    hi_u = lax.bitcast_convert_type(hi.astype(jnp.bfloat16),
                                    jnp.uint16).astype(jnp.uint32)
    return lax.bitcast_convert_type(lo_u | (hi_u << 16), jnp.float32)


def _unpack2(w):
    wu = lax.bitcast_convert_type(w, jnp.uint32)
    lo = lax.bitcast_convert_type((wu & 0xFFFF).astype(jnp.uint16),
                                  jnp.bfloat16).astype(jnp.float32)
    hi = lax.bitcast_convert_type((wu >> 16).astype(jnp.uint16),
                                  jnp.bfloat16).astype(jnp.float32)
    return lo, hi


def _silu(x):
    return x * jax.nn.sigmoid(x)


def _dot(a, b):
    return jnp.dot(a, b, preferred_element_type=jnp.float32)


def _prep_weights(params):
    gs = [('Wp_edge', 'bp_edge', 'blk_p'), ('Wl_edge', 'bl_edge', 'blk_l'),
          ('Wc_edge', 'bc_edge', 'blk_c')]
    ins = []
    for wk, bk, blk in gs:
        ins += [params[wk], params[bk].reshape(1, _D),
                params[blk]['We1'], params[blk]['be1'],
                params[blk]['bx'].reshape(2, 1),
                params[blk]['Wx'].reshape(2, 1, _D),
                params[blk]['be2']]

    def body(*refs):
        irefs, orefs = refs[:21], refs[21:]
        for g in range(3):
            (we_r, be_r, we1_r, be1_r, bx_r, wx_r,
             be2_r) = irefs[7 * g:7 * g + 7]
            o_r = orefs[g]
            for i in range(2):
                wmat = we1_r[i, 2 * _D + 1:, :]
                o_r[i, 0:16, :] = _dot(we_r[...], wmat)
                o_r[i, 16:17, :] = _dot(be_r[...], wmat) + be1_r[i:i + 1, :]
                o_r[i, 17:18, :] = we1_r[i, 2 * _D:2 * _D + 1, :]
                o_r[i, 18:19, :] = jnp.broadcast_to(bx_r[i:i + 1, :], (1, _D))
                o_r[i, 19:20, :] = wx_r[i]
                o_r[i, 20:21, :] = be2_r[i:i + 1, :]
                o_r[i, 21:24, :] = jnp.zeros((3, _D), jnp.float32)

    out_shape = [jax.ShapeDtypeStruct((2, 24, _D), jnp.float32)] * 3
    return pl.pallas_call(body, out_shape=out_shape)(*ins)


def _init_node(x, W2, ext2, split):
    N, F = x.shape

    def body(x_r, w_r, e_r, o_r):
        h = _dot(x_r[...], w_r[0]) + e_r[0, 0:1, :]
        mu = jnp.mean(h, axis=-1, keepdims=True)
        hm = h - mu
        v = jnp.mean(hm * hm, axis=-1, keepdims=True)
        o_r[...] = (hm * lax.rsqrt(v + 1e-5) * e_r[0, 1:2, :] +
                    e_r[0, 2:3, :])

    return pl.pallas_call(
        body,
        grid=(N // _BN,),
        in_specs=[pl.BlockSpec((_BN, F), lambda i: (i, 0)),
                  pl.BlockSpec((1, F, _D), lambda i: (i // split, 0, 0)),
                  pl.BlockSpec((1, 8, _D), lambda i: (i // split, 0, 0))],
        out_specs=pl.BlockSpec((_BN, _D), lambda i: (i, 0)),
        out_shape=jax.ShapeDtypeStruct((N, _D), jnp.float32),
    )(x, W2, ext2)


def _ab_prep(h, xpad, whd2, whs2, split):
    N = h.shape[0]

    def body(h_r, x_r, a_w, b_w, a_o, b_o):
        hv = h_r[...]
        xv = x_r[...]
        zx = jnp.zeros((_BN, 128 - _XW), jnp.float32)
        for o_r, w_r in ((a_o, a_w), (b_o, b_w)):
            av = _dot(hv, w_r[0])
            o_r[...] = jnp.concatenate(
                [_pack2(av[:, :128], av[:, 128:]), xv, zx], axis=-1)

    return pl.pallas_call(
        body,
        grid=(N // _BN,),
        in_specs=[pl.BlockSpec((_BN, _D), lambda i: (i, 0)),
                  pl.BlockSpec((_BN, _XW), lambda i: (i, 0)),
                  pl.BlockSpec((1, _D, _D), lambda i: (i // split, 0, 0)),
                  pl.BlockSpec((1, _D, _D), lambda i: (i // split, 0, 0))],
        out_specs=[pl.BlockSpec((_BN, _TW), lambda i: (i, 0))] * 2,
        out_shape=[jax.ShapeDtypeStruct((N, _TW), jnp.float32)] * 2,
    )(h, xpad, whd2, whs2)


def _edge_mlp(ga, gb, xd, xs, feats, wext2, we22, split, with_coef):
    E = ga.shape[0]

    def body(ga_r, gb_r, xd_r, xs_r, ft_r, wext_r, we2_r, *outs):
        ga0, ga1 = _unpack2(ga_r[...])
        gb0, gb1 = _unpack2(gb_r[...])
        rel = xd_r[...] - xs_r[...]
        d2 = jnp.sum(rel * rel, axis=-1, keepdims=True)
        wc = wext_r[0, 0:16, :]
        bc = wext_r[0, 16:17, :]
        wd2 = wext_r[0, 17:18, :]
        gsum = jnp.concatenate([ga0 + gb0, ga1 + gb1], axis=-1)
        pre = gsum + _dot(ft_r[...], wc) + bc + d2 * wd2
        m1 = _silu(pre)
        m = _silu(_dot(m1, we2_r[0]) + wext_r[0, 20:21, :])
        outs[0][0, :, :] = m[:, :128]
        outs[0][1, :, :] = m[:, 128:]
        if with_coef:
            bx = wext_r[0, 18:19, 0:1]
            wx = wext_r[0, 19:20, :]
            coef = jnp.sum(m * wx, axis=-1, keepdims=True) + bx
            outs[1][...] = jnp.concatenate(
                [rel * coef, jnp.zeros((_BE, 128 - _XW), jnp.float32)],
                axis=-1)

    out_shape = [jax.ShapeDtypeStruct((2, E, 128), jnp.float32)]
    out_specs = [pl.BlockSpec((2, _BE, 128), lambda i: (0, i, 0))]
    if with_coef:
        out_shape.append(jax.ShapeDtypeStruct((E, 128), jnp.float32))
        out_specs.append(pl.BlockSpec((_BE, 128), lambda i: (i, 0)))

    return pl.pallas_call(
        body,
        grid=(E // _BE,),
        in_specs=[pl.BlockSpec((_BE, 128), lambda i: (i, 0)),
                  pl.BlockSpec((_BE, 128), lambda i: (i, 0)),
                  pl.BlockSpec((_BE, _XW), lambda i: (i, 0)),
                  pl.BlockSpec((_BE, _XW), lambda i: (i, 0)),
                  pl.BlockSpec((_BE, 16), lambda i: (i, 0)),
                  pl.BlockSpec((1, 24, _D), lambda i: (i // split, 0, 0)),
                  pl.BlockSpec((1, _D, _D), lambda i: (i // split, 0, 0))],
        out_specs=out_specs,
        out_shape=out_shape,
    )(ga, gb, xd, xs, feats, wext2, we22)


def _node_update(h, ag2s, w1a2, w1ba2, w1bb2, w22, next2, x, dx2s, split,
                 with_x):
    N = h.shape[0]
    na = len(ag2s)
    nd = len(dx2s) if with_x else 0

    def body(*refs):
        h_r = refs[0]
        ag_rs = refs[1:1 + na]
        w1a_r, w1ba_r, w1bb_r, w2_r, ne_r = refs[1 + na:6 + na]
        if with_x:
            x_r = refs[6 + na]
            dx_rs = refs[7 + na:7 + na + nd]
            ho_r, xo_r = refs[7 + na + nd:]
        else:
            ho_r = refs[6 + na]
        hv = h_r[...]
        aga = ag_rs[0][0, :, :]
        agb = ag_rs[0][1, :, :]
        for r in ag_rs[1:]:
            aga = aga + r[0, :, :]
            agb = agb + r[1, :, :]
        t = (_dot(hv, w1a_r[0]) + _dot(aga, w1ba_r[0]) +
             _dot(agb, w1bb_r[0]) + ne_r[0, 0:1, :])
        t = _silu(t)
        ho_r[...] = hv + _dot(t, w2_r[0]) + ne_r[0, 1:2, :]
        if with_x:
            dx = dx_rs[0][0, :, :] + dx_rs[0][1, :, :]
            for r in dx_rs[1:]:
                dx = dx + r[0, :, :] + r[1, :, :]
            xo_r[...] = x_r[...] + dx[:, :_XW] * (1.0 / 16.0)

    agspec = pl.BlockSpec((2, _BN, 128), lambda i: (0, i, 0))
    in_specs = ([pl.BlockSpec((_BN, _D), lambda i: (i, 0))] +
                [agspec] * na +
                [pl.BlockSpec((1, _D, _D), lambda i: (i // split, 0, 0)),
                 pl.BlockSpec((1, 128, _D), lambda i: (i // split, 0, 0)),
                 pl.BlockSpec((1, 128, _D), lambda i: (i // split, 0, 0)),
                 pl.BlockSpec((1, _D, _D), lambda i: (i // split, 0, 0)),
                 pl.BlockSpec((1, 8, _D), lambda i: (i // split, 0, 0))])
    out_specs = [pl.BlockSpec((_BN, _D), lambda i: (i, 0))]
    out_shape = [jax.ShapeDtypeStruct((N, _D), jnp.float32)]
    args = [h] + list(ag2s) + [w1a2, w1ba2, w1bb2, w22, next2]
    if with_x:
        in_specs += [pl.BlockSpec((_BN, _XW), lambda i: (i, 0))]
        in_specs += [agspec] * nd
        out_specs.append(pl.BlockSpec((_BN, _XW), lambda i: (i, 0)))
        out_shape.append(jax.ShapeDtypeStruct((N, _XW), jnp.float32))
        args += [x] + list(dx2s)

    res = pl.pallas_call(
        body, grid=(N // _BN,), in_specs=in_specs, out_specs=out_specs,
        out_shape=out_shape)(*args)
    return res if with_x else (res[0], None)


def _sc_gather(td, ts, src, dst):
    E = src.shape[0]
    ept = E // _NW
    CG = 64
    nfull = ept // CG
    ctail = ept - nfull * CG
    npairs = nfull // 2
    rem = nfull % 2
    mesh = plsc.VectorSubcoreMesh(core_axis_name="c", subcore_axis_name="s")

    @functools.partial(
        pl.kernel, mesh=mesh,
        out_type=[jax.ShapeDtypeStruct((E, 128), jnp.float32),
                  jax.ShapeDtypeStruct((E, 128), jnp.float32),
                  jax.ShapeDtypeStruct((E, _XW), jnp.float32),
                  jax.ShapeDtypeStruct((E, _XW), jnp.float32)],
        scratch_types=[pltpu.VMEM((CG,), jnp.int32),
                       pltpu.VMEM((CG,), jnp.int32),
                       pltpu.VMEM((CG,), jnp.int32),
                       pltpu.VMEM((CG,), jnp.int32),
                       pltpu.VMEM((CG, _TW), jnp.float32),
                       pltpu.VMEM((CG, _TW), jnp.float32),
                       pltpu.VMEM((CG, _TW), jnp.float32),
                       pltpu.VMEM((CG, _TW), jnp.float32),
                       pltpu.VMEM((CG, _XW), jnp.float32),
                       pltpu.VMEM((CG, _XW), jnp.float32),
                       pltpu.VMEM((CG, _XW), jnp.float32),
                       pltpu.VMEM((CG, _XW), jnp.float32),
                       pltpu.SemaphoreType.DMA, pltpu.SemaphoreType.DMA,
                       pltpu.SemaphoreType.DMA, pltpu.SemaphoreType.DMA,
                       pltpu.SemaphoreType.DMA, pltpu.SemaphoreType.DMA])
    def k(a_hbm, b_hbm, src_hbm, dst_hbm, ga_hbm, gb_hbm, xd_hbm, xs_hbm,
          si0, di0, si1, di1, ab0, bb0, ab1, bb1,
          xd0, xs0, xd1, xs1,
          sa0, sb0, sa1, sb1, sw0, sw1):
        wid = lax.axis_index("s") * _NCORE + lax.axis_index("c")
        base = wid * ept

        def fetch(eb, n, si, di, ab, bb, sa, sb):
            sin = si.at[pl.ds(0, n)] if n != CG else si
            din = di.at[pl.ds(0, n)] if n != CG else di
            abn = ab.at[pl.ds(0, n)] if n != CG else ab
            bbn = bb.at[pl.ds(0, n)] if n != CG else bb
            pltpu.sync_copy(src_hbm.at[pl.ds(eb, n)], sin)
            pltpu.sync_copy(dst_hbm.at[pl.ds(eb, n)], din)
            ca = pltpu.async_copy(a_hbm.at[din], abn, sa)
            cb = pltpu.async_copy(b_hbm.at[sin], bbn, sb)
            return ca, cb

        def wback(eb, n, ab, bb, xdb, xsb, sw):
            @pl.loop(0, n)
            def _(r):
                xdb[r, pl.ds(0, _XW)] = ab[r, pl.ds(128, _XW)]
                xsb[r, pl.ds(0, _XW)] = bb[r, pl.ds(128, _XW)]

            es = pl.ds(eb, n)
            rs = pl.ds(0, n)
            xdn = xdb.at[rs] if n != CG else xdb
            xsn = xsb.at[rs] if n != CG else xsb
            return [
                pltpu.async_copy(ab.at[rs, pl.ds(0, 128)],
                                 ga_hbm.at[es], sw),
                pltpu.async_copy(bb.at[rs, pl.ds(0, 128)],
                                 gb_hbm.at[es], sw),
                pltpu.async_copy(xdn, xd_hbm.at[es], sw),
                pltpu.async_copy(xsn, xs_hbm.at[es], sw),
            ]

        @pl.loop(0, npairs)
        def _(j):
            e0 = base + j * (2 * CG)
            e1 = e0 + CG
            ca0, cb0 = fetch(e0, CG, si0, di0, ab0, bb0, sa0, sb0)
            ca1, cb1 = fetch(e1, CG, si1, di1, ab1, bb1, sa1, sb1)
            ca0.wait()
            cb0.wait()
            w0 = wback(e0, CG, ab0, bb0, xd0, xs0, sw0)
            ca1.wait()
            cb1.wait()
            w1 = wback(e1, CG, ab1, bb1, xd1, xs1, sw1)
            for w in w0 + w1:
                w.wait()

        tb = base + npairs * 2 * CG
        if rem:
            ca, cb = fetch(tb, CG, si0, di0, ab0, bb0, sa0, sb0)
            ca.wait()
            cb.wait()
            for w in wback(tb, CG, ab0, bb0, xd0, xs0, sw0):
                w.wait()
            tb = tb + CG
        if ctail:
            ca, cb = fetch(tb, ctail, si1, di1, ab1, bb1, sa1, sb1)
            ca.wait()
            cb.wait()
            for w in wback(tb, ctail, ab1, bb1, xd1, xs1, sw1):
                w.wait()

    return k(td, ts, src, dst)


def _sc_scatter_m(m2, dst, N):
    E = dst.shape[0]
    ept = E // _NSUB
    CS = 64
    nfull = ept // CS
    ctail = ept - nfull * CS
    npairs = nfull // 2
    rem = nfull % 2
    rpt = (N // _NSUB) // 8 * 8
    tail = N - _NSUB * rpt
    mesh = plsc.VectorSubcoreMesh(core_axis_name="c", subcore_axis_name="s")
    z128 = jnp.zeros((N, 128), jnp.float32)

    @functools.partial(
        pl.kernel, mesh=mesh,
        out_type=jax.ShapeDtypeStruct((2, N, 128), jnp.float32),
        scratch_types=[pltpu.VMEM((CS,), jnp.int32),
                       pltpu.VMEM((CS,), jnp.int32),
                       pltpu.VMEM((ctail or 8,), jnp.int32),
                       pltpu.VMEM((CS, 128), jnp.float32),
                       pltpu.VMEM((CS, 128), jnp.float32),
                       pltpu.VMEM((ctail or 8, 128), jnp.float32),
                       pltpu.VMEM_SHARED((N, 128), jnp.float32),
                       pltpu.SemaphoreType.DMA, pltpu.SemaphoreType.DMA,
                       pltpu.SemaphoreType.DMA, pltpu.SemaphoreType.DMA])
    def k(m_hbm, dst_hbm, z_hbm, agg_hbm,
          di0, di1, dit, rows0, rows1, rowst, acc, sr0, sr1, sd0, sd1):
        cid = lax.axis_index("c")
        sid = lax.axis_index("s")

        @pl.when(sid == 0)
        def _():
            pltpu.sync_copy(z_hbm, acc)

        plsc.subcore_barrier()
        base = sid * ept

        def fetch(eb, n, di, rows, sr):
            pltpu.sync_copy(dst_hbm.at[pl.ds(eb, n)], di)
            cr = pltpu.async_copy(m_hbm.at[cid, pl.ds(eb, n)], rows, sr)
            return cr, di, rows

        @pl.loop(0, npairs)
        def _(j):
            e0 = base + j * (2 * CS)
            c0, d0, r0 = fetch(e0, CS, di0, rows0, sr0)
            c1, d1, r1 = fetch(e0 + CS, CS, di1, rows1, sr1)
            c0.wait()
            a0 = pltpu.async_copy(r0, acc.at[d0], sd0, add=True)
            c1.wait()
            a1 = pltpu.async_copy(r1, acc.at[d1], sd1, add=True)
            a0.wait()
            a1.wait()

        tb = base + npairs * 2 * CS
        if rem:
            c0, d0, r0 = fetch(tb, CS, di0, rows0, sr0)
            c0.wait()
            pltpu.async_copy(r0, acc.at[d0], sd0, add=True).wait()
            tb = tb + CS
        if ctail:
            c1, d1, r1 = fetch(tb, ctail, dit, rowst, sr1)
            c1.wait()
            pltpu.async_copy(r1, acc.at[d1], sd1, add=True).wait()

        plsc.subcore_barrier()
        rb = sid * rpt
        pltpu.sync_copy(acc.at[pl.ds(rb, rpt)],
                        agg_hbm.at[cid, pl.ds(rb, rpt)])
        if tail:
            @pl.when(sid == 0)
            def _():
                tn = _NSUB * rpt
                pltpu.sync_copy(acc.at[pl.ds(tn, tail)],
                                agg_hbm.at[cid, pl.ds(tn, tail)])

    return k(m2, dst, z128)


def _sc_scatter_rc(rc, dst, N):
    E = dst.shape[0]
    eph = E // 2
    ept = eph // _NSUB
    CS = 64
    nfull = ept // CS
    ctail = ept - nfull * CS
    npairs = nfull // 2
    rem = nfull % 2
    rpt = (N // _NSUB) // 8 * 8
    tail = N - _NSUB * rpt
    mesh = plsc.VectorSubcoreMesh(core_axis_name="c", subcore_axis_name="s")
    z128 = jnp.zeros((N, 128), jnp.float32)

    @functools.partial(
        pl.kernel, mesh=mesh,
        out_type=jax.ShapeDtypeStruct((2, N, 128), jnp.float32),
        scratch_types=[pltpu.VMEM((CS,), jnp.int32),
                       pltpu.VMEM((CS,), jnp.int32),
                       pltpu.VMEM((ctail or 8,), jnp.int32),
                       pltpu.VMEM((CS, 128), jnp.float32),
                       pltpu.VMEM((CS, 128), jnp.float32),
                       pltpu.VMEM((ctail or 8, 128), jnp.float32),
                       pltpu.VMEM_SHARED((N, 128), jnp.float32),
                       pltpu.SemaphoreType.DMA, pltpu.SemaphoreType.DMA,
                       pltpu.SemaphoreType.DMA, pltpu.SemaphoreType.DMA])
    def k(rc_hbm, dst_hbm, z_hbm, dx_hbm,
          di0, di1, dit, rows0, rows1, rowst, acc, sr0, sr1, sd0, sd1):
        cid = lax.axis_index("c")
        sid = lax.axis_index("s")

        @pl.when(sid == 0)
        def _():
            pltpu.sync_copy(z_hbm, acc)

        plsc.subcore_barrier()
        base = cid * eph + sid * ept

        def fetch(eb, n, di, rows, sr):
            pltpu.sync_copy(dst_hbm.at[pl.ds(eb, n)], di)
            cr = pltpu.async_copy(rc_hbm.at[pl.ds(eb, n)], rows, sr)
            return cr, di, rows

        @pl.loop(0, npairs)
        def _(j):
            e0 = base + j * (2 * CS)
            c0, d0, r0 = fetch(e0, CS, di0, rows0, sr0)
            c1, d1, r1 = fetch(e0 + CS, CS, di1, rows1, sr1)
            c0.wait()
            a0 = pltpu.async_copy(r0, acc.at[d0], sd0, add=True)
            c1.wait()
            a1 = pltpu.async_copy(r1, acc.at[d1], sd1, add=True)
            a0.wait()
            a1.wait()

        tb = base + npairs * 2 * CS
        if rem:
            c0, d0, r0 = fetch(tb, CS, di0, rows0, sr0)
            c0.wait()
            pltpu.async_copy(r0, acc.at[d0], sd0, add=True).wait()
            tb = tb + CS
        if ctail:
            c1, d1, r1 = fetch(tb, ctail, dit, rowst, sr1)
            c1.wait()
            pltpu.async_copy(r1, acc.at[d1], sd1, add=True).wait()

        plsc.subcore_barrier()
        rb = sid * rpt
        pltpu.sync_copy(acc.at[pl.ds(rb, rpt)],
                        dx_hbm.at[cid, pl.ds(rb, rpt)])
        if tail:
            @pl.when(sid == 0)
            def _():
                tn = _NSUB * rpt
                pltpu.sync_copy(acc.at[pl.ds(tn, tail)],
                                dx_hbm.at[cid, pl.ds(tn, tail)])

    return k(rc, dst, z128)


def _egnn_block(h, xpad, feats, src, dst, stk, wext2, split_n, split_e,
                with_x, parts=None):
    N = h.shape[0]
    E = src.shape[0]
    parts = parts or [E]
    td, ts = _ab_prep(h, xpad, stk['whd'], stk['whs'], split_n)
    ag2s, dx2s = [], []
    off = 0
    for Eh in parts:
        s_ = src[off:off + Eh]
        d_ = dst[off:off + Eh]
        f_ = feats[off:off + Eh]
        off += Eh
        ga, gb, xd, xs = _sc_gather(td, ts, s_, d_)
        res = _edge_mlp(ga, gb, xd, xs, f_, wext2, stk['we2'], split_e,
                        with_coef=with_x)
        ag2s.append(_sc_scatter_m(res[0], d_, N))
        if with_x:
            dx2s.append(_sc_scatter_rc(res[1], d_, N))
    return _node_update(h, ag2s, stk['w1a'], stk['w1ba'], stk['w1bb'],
                        stk['w2'], stk['next'], xpad, dx2s, split_n, with_x)


def _stack_layer(blks, i):
    z6 = jnp.zeros((6, _D), jnp.float32)
    return {
        'whd': jnp.stack([b['We1'][i, :_D, :] for b in blks]),
        'whs': jnp.stack([b['We1'][i, _D:2 * _D, :] for b in blks]),
        'we2': jnp.stack([b['We2'][i] for b in blks]),
        'w1a': jnp.stack([b['Wh1'][i, :_D, :] for b in blks]),
        'w1ba': jnp.stack([b['Wh1'][i, _D:_D + 128, :] for b in blks]),
        'w1bb': jnp.stack([b['Wh1'][i, _D + 128:, :] for b in blks]),
        'w2': jnp.stack([b['Wh2'][i] for b in blks]),
        'next': jnp.stack([jnp.concatenate(
            [b['bh1'][i][None], b['bh2'][i][None], z6]) for b in blks]),
    }


def kernel(xp, edge_index_p, ep_feats, coord_p, xl, edge_index_l, el_feats,
           coord_l, edge_index_c, ec_feats, coord_c, params):
    NP = xp.shape[0]
    L = params['blk_p']['We1'].shape[0]
    BIG = 1 << 20

    def pad_x(c):
        n = c.shape[0]
        return jnp.concatenate(
            [c, jnp.zeros((n, _XW - c.shape[1]), jnp.float32)], axis=1)

    sp, dp = (edge_index_p[0].astype(jnp.int32),
              edge_index_p[1].astype(jnp.int32))
    sl, dl = (edge_index_l[0].astype(jnp.int32),
              edge_index_l[1].astype(jnp.int32))
    sc, dc = (edge_index_c[0].astype(jnp.int32),
              edge_index_c[1].astype(jnp.int32))

    xpp, xpl, xpc = pad_x(coord_p), pad_x(coord_l), pad_x(coord_c)

    wext_p, wext_l, wext_c = _prep_weights(params)
    z5 = jnp.zeros((5, _D), jnp.float32)

    def init_ext(b, g, bb):
        return jnp.concatenate([b[None], g[None], bb[None], z5])

    hp = _init_node(xp, params['Wp_node'][None],
                    init_ext(params['bp_node'], params['ln_p_g'],
                             params['ln_p_b'])[None], BIG)
    hl = _init_node(xl, params['Wl_node'][None],
                    init_ext(params['bl_node'], params['ln_l_g'],
                             params['ln_l_b'])[None], BIG)

    for i in range(L):
        with_x = i < L - 1
        stk_p = _stack_layer([params['blk_p']], i)
        stk_l = _stack_layer([params['blk_l']], i)
        stk_c = _stack_layer([params['blk_c']], i)
        hp, xpp = _egnn_block(hp, xpp, ep_feats, sp, dp, stk_p,
                              wext_p[i][None], BIG, BIG, with_x,
                              parts=[64000, 96000])
        hl, xpl = _egnn_block(hl, xpl, el_feats, sl, dl, stk_l,
                              wext_l[i][None], BIG, BIG, with_x)
        hc = jnp.concatenate([hp, hl], axis=0)
        hc, xpc = _egnn_block(hc, xpc, ec_feats, sc, dc, stk_c,
                              wext_c[i][None], BIG, BIG, with_x,
                              parts=[96000, 96000])
        hp = hc[:NP]
        hl = hc[NP:]

    return hp, hl, hc
